# baseline (speedup 1.0000x reference)
import functools

import jax
import jax.numpy as jnp
from jax import lax
from jax.experimental import pallas as pl
from jax.experimental.pallas import tpu as pltpu
from jax.experimental.pallas import tpu_sc as plsc

N = 2048
C = 1024
E = 8
I = 2816
K = 2
BLK = 512
IBLK = 256
NI = I // IBLK
NSLOT = N * K + E * BLK
NBLK = NSLOT // BLK
DUMP = NSLOT
NWSLOT = NSLOT + 8
NPAD = E * BLK

_NC, _NS = 2, 16
_NW = _NC * _NS
_PPW = (N * K) // _NW
_TPW = N // _NW
_SPW = NSLOT // _NW

_INTERPRET = False


def _dispatch_kernel(x_ref, wg_ref, pos1_ref, pos2_ref, p1_ref, p2_ref,
                     pad_ref, be_ref, valid_ref):
    x = x_ref[...]
    logits = lax.dot_general(x, wg_ref[...], (((1,), (1,)), ((), ())),
                             preferred_element_type=jnp.float32)
    e_iota = lax.broadcasted_iota(jnp.int32, (N, E), 1)
    i1 = jnp.argmax(logits, axis=1)[:, None]
    m1 = jnp.max(logits, axis=1, keepdims=True)
    masked = jnp.where(e_iota == i1, -jnp.inf, logits)
    i2 = jnp.argmax(masked, axis=1)[:, None]
    m2 = jnp.max(masked, axis=1, keepdims=True)
    e2 = jnp.exp(m2 - m1)
    p1_ref[...] = 1.0 / (1.0 + e2)
    p2_ref[...] = e2 / (1.0 + e2)

    M1 = (e_iota == i1).astype(jnp.float32)
    M2 = (e_iota == i2).astype(jnp.float32)
    M12 = jnp.concatenate([M1, M2], axis=1).astype(jnp.bfloat16)
    r_iota = lax.broadcasted_iota(jnp.int32, (N, N), 0)
    c_iota = lax.broadcasted_iota(jnp.int32, (N, N), 1)
    T = (c_iota < r_iota).astype(jnp.bfloat16)
    R12 = lax.dot_general(T, M12, (((1,), (0,)), ((), ())),
                          preferred_element_type=jnp.float32)
    cnt1 = jnp.sum(M1, axis=0, keepdims=True)
    cnt2 = jnp.sum(M2, axis=0, keepdims=True)
    counts = cnt1 + cnt2
    rank1 = R12[:, :E]
    rank2 = cnt1 + R12[:, E:]
    pc = jnp.floor((counts + (BLK - 1)) * (1.0 / BLK)) * BLK
    pb = jnp.zeros((1, E), jnp.float32)
    for k in range(1, E):
        pb = pb + jnp.concatenate(
            [jnp.zeros((1, k), jnp.float32), pc[:, :E - k]], axis=1)
    pos1_ref[...] = jnp.sum(M1 * (pb + rank1), axis=1, keepdims=True)
    pos2_ref[...] = jnp.sum(M2 * (pb + rank2), axis=1, keepdims=True)

    r8 = lax.broadcasted_iota(jnp.int32, (E, BLK), 0)
    j8 = lax.broadcasted_iota(jnp.int32, (E, BLK), 1).astype(jnp.float32)
    padpos = jnp.zeros((E, BLK), jnp.float32)
    for e in range(E):
        base_e = pb[0, e] + counts[0, e]
        npad_e = pc[0, e] - counts[0, e]
        padpos = padpos + jnp.where(
            r8 == e,
            jnp.where(j8 < npad_e, base_e + j8, jnp.float32(DUMP)),
            0.0)
    pad_ref[...] = padpos

    biota = lax.broadcasted_iota(jnp.int32, (1, 128), 1).astype(jnp.float32)
    beval = jnp.zeros((1, 128), jnp.float32)
    for e in range(1, E):
        beval = beval + (biota >= pb[0, e] * (1.0 / BLK)).astype(jnp.float32)
    be_ref[...] = beval.astype(jnp.int32)
    used_blocks = jnp.sum(pc) * (1.0 / BLK)
    valid_ref[...] = (biota < used_blocks).astype(jnp.int32)


def _sc_dispatch_body(x_hbm, pos_hbm, p_hbm, pad_hbm, xs_hbm, w_hbm,
                      posf_v, posi_v, posr_v, pv_v, padf_v, padi_v, zero_v,
                      rows_a, rows_b, semr_a, semr_b, semw_a, semw_b, sem):
    wid = lax.axis_index("s") * _NC + lax.axis_index("c")
    base = wid * _PPW
    tokbase = (wid % _NS) * _PPW
    npad_w = NPAD // _NW
    pltpu.sync_copy(pos_hbm.at[pl.ds(base, _PPW)], posf_v)
    pltpu.sync_copy(p_hbm.at[pl.ds(base, _PPW)], pv_v)
    pltpu.sync_copy(pad_hbm.at[pl.ds(wid * npad_w, npad_w)], padf_v)
    for j in range(4):
        for i in range(2):
            off = 32 * j + 16 * i
            v = posf_v[pl.ds(off, 16)].astype(jnp.int32)
            posr_v[pl.ds(off, 16)] = v
            posi_v[j, pl.ds(16 * i, 16)] = v
    for i in range(npad_w // 16):
        padi_v[pl.ds(16 * i, 16)] = padf_v[pl.ds(16 * i, 16)].astype(jnp.int32)
        zero_v[pl.ds(16 * i, 16)] = jnp.zeros((16,), jnp.float32)
    cw = pltpu.async_copy(pv_v, w_hbm.at[posr_v], sem)
    cz = pltpu.async_copy(zero_v, w_hbm.at[padi_v], sem)
    bufs = (rows_a, rows_b)
    semr = (semr_a, semr_b)
    semw = (semw_a, semw_b)
    rd = [None] * 4
    wr = [None] * 4
    rd[0] = pltpu.async_copy(x_hbm.at[pl.ds(tokbase, 32), :], rows_a, semr_a)
    for j in range(4):
        rd[j].wait()
        if j + 1 < 4:
            if wr[j - 1] is not None:
                wr[j - 1].wait()
            rd[j + 1] = pltpu.async_copy(
                x_hbm.at[pl.ds(tokbase + 32 * (j + 1), 32), :],
                bufs[(j + 1) % 2], semr[(j + 1) % 2])
        wr[j] = pltpu.async_copy(bufs[j % 2], xs_hbm.at[posi_v.at[j]],
                                 semw[j % 2])
    wr[2].wait()
    wr[3].wait()
    cw.wait()
    cz.wait()


@functools.cache
def _get_sc_kernels():
    sc_mesh = plsc.VectorSubcoreMesh(core_axis_name="c", subcore_axis_name="s",
                                     num_cores=_NC, num_subcores=_NS)
    npad_w = NPAD // _NW
    sc_dispatch = functools.partial(
        pl.kernel,
        out_type=(jax.ShapeDtypeStruct((NSLOT, C), jnp.float32),
                  jax.ShapeDtypeStruct((NWSLOT,), jnp.float32)),
        mesh=sc_mesh,
        scratch_types=[
            pltpu.VMEM((_PPW,), jnp.float32),
            pltpu.VMEM((4, 32), jnp.int32),
            pltpu.VMEM((_PPW,), jnp.int32),
            pltpu.VMEM((_PPW,), jnp.float32),
            pltpu.VMEM((npad_w,), jnp.float32),
            pltpu.VMEM((npad_w,), jnp.int32),
            pltpu.VMEM((npad_w,), jnp.float32),
            pltpu.VMEM((32, C), jnp.float32),
            pltpu.VMEM((32, C), jnp.float32),
            pltpu.SemaphoreType.DMA,
            pltpu.SemaphoreType.DMA,
            pltpu.SemaphoreType.DMA,
            pltpu.SemaphoreType.DMA,
            pltpu.SemaphoreType.DMA,
        ],
    )(_sc_dispatch_body)
    sc_combine = functools.partial(
        pl.kernel,
        out_type=jax.ShapeDtypeStruct((N, C), jnp.float32),
        mesh=sc_mesh,
        scratch_types=[
            pltpu.VMEM((_TPW,), jnp.float32),
            pltpu.VMEM((2, 32), jnp.int32),
            pltpu.VMEM((2, 32), jnp.int32),
            pltpu.VMEM((32, C), jnp.float32),
            pltpu.VMEM((32, C), jnp.float32),
            pltpu.SemaphoreType.DMA,
            pltpu.SemaphoreType.DMA,
        ],
    )(_sc_combine_body)
    return sc_dispatch, sc_combine


def _moe_sparse_kernel(be_ref, valid_ref, ws_ref, xs_ref, w1_ref, w2_ref,
                       wp_ref, out_ref):
    i = pl.program_id(0)
    b = pl.program_id(1)

    @pl.when(valid_ref[b] == 1)
    def _compute():
        rows = xs_ref[...].astype(jnp.bfloat16)
        w1 = w1_ref[0].astype(jnp.bfloat16)
        w2 = w2_ref[0].astype(jnp.bfloat16)
        wp = wp_ref[0].astype(jnp.bfloat16)
        h1 = lax.dot_general(rows, w1, (((1,), (1,)), ((), ())),
                             preferred_element_type=jnp.float32)
        h2 = lax.dot_general(rows, w2, (((1,), (1,)), ((), ())),
                             preferred_element_type=jnp.float32)
        h = h1 * jax.nn.sigmoid(h1) * h2
        hw = (h * ws_ref[...]).astype(jnp.bfloat16)
        pout = lax.dot_general(hw, wp, (((1,), (1,)), ((), ())),
                               preferred_element_type=jnp.float32)
        sl = pl.ds(pl.multiple_of(b * BLK, BLK), BLK)

        @pl.when(i == 0)
        def _first():
            out_ref[sl, :] = pout

        @pl.when(i > 0)
        def _acc():
            out_ref[sl, :] += pout


def _sc_combine_body(ys_hbm, pos1_hbm, pos2_hbm, y_hbm,
                     pf_v, pia_v, pib_v, bufa_v, bufb_v, sema, semb):
    wid = lax.axis_index("s") * _NC + lax.axis_index("c")
    tbase = wid * _TPW
    pltpu.sync_copy(pos1_hbm.at[pl.ds(tbase, _TPW)], pf_v)
    for j in range(2):
        for i in range(2):
            pia_v[j, pl.ds(16 * i, 16)] = (
                pf_v[pl.ds(32 * j + 16 * i, 16)].astype(jnp.int32))
    pltpu.sync_copy(pos2_hbm.at[pl.ds(tbase, _TPW)], pf_v)
    for j in range(2):
        for i in range(2):
            pib_v[j, pl.ds(16 * i, 16)] = (
                pf_v[pl.ds(32 * j + 16 * i, 16)].astype(jnp.int32))
    for j in range(2):
        ca = pltpu.async_copy(ys_hbm.at[pia_v.at[j]], bufa_v, sema)
        cb = pltpu.async_copy(ys_hbm.at[pib_v.at[j]], bufb_v, semb)
        ca.wait()
        cb.wait()

        def row_body(r, carry):
            for v in range(C // 16):
                s = pl.ds(pl.multiple_of(16 * v, 16), 16)
                bufa_v[r, s] = bufa_v[r, s] + bufb_v[r, s]
            return carry

        lax.fori_loop(0, 32, row_body, 0)
        pltpu.sync_copy(bufa_v, y_hbm.at[pl.ds(tbase + 32 * j, 32), :])


def kernel(x, Wg, W1, W2, Wp):
    Bv, Tv, Cv = x.shape
    xf = x.reshape(N, C)

    pos1, pos2, p1, p2, padpos, be, valid = pl.pallas_call(
        _dispatch_kernel,
        out_shape=(
            jax.ShapeDtypeStruct((N, 1), jnp.float32),
            jax.ShapeDtypeStruct((N, 1), jnp.float32),
            jax.ShapeDtypeStruct((N, 1), jnp.float32),
            jax.ShapeDtypeStruct((N, 1), jnp.float32),
            jax.ShapeDtypeStruct((E, BLK), jnp.float32),
            jax.ShapeDtypeStruct((1, 128), jnp.int32),
            jax.ShapeDtypeStruct((1, 128), jnp.int32),
        ),
        interpret=_INTERPRET,
    )(xf, Wg)

    pos_flat = jnp.concatenate([pos1.reshape(-1), pos2.reshape(-1)])
    p_flat = jnp.concatenate([p1.reshape(-1), p2.reshape(-1)])
    pad_flat = padpos.reshape(-1)

    sc_dispatch, sc_combine = _get_sc_kernels()
    xs, wraw = sc_dispatch(xf, pos_flat, p_flat, pad_flat)
    ws2d = wraw[:NSLOT].reshape(NSLOT, 1)
    be_flat = be.reshape(-1)[:NBLK]
    valid_flat = valid.reshape(-1)[:NBLK]

    grid_spec = pltpu.PrefetchScalarGridSpec(
        num_scalar_prefetch=2,
        grid=(NI, NBLK),
        in_specs=[
            pl.BlockSpec((BLK, 1), lambda i, b, be, va: (b, 0)),
            pl.BlockSpec((BLK, C), lambda i, b, be, va: (b, 0)),
            pl.BlockSpec((1, IBLK, C), lambda i, b, be, va: (be[b], i, 0)),
            pl.BlockSpec((1, IBLK, C), lambda i, b, be, va: (be[b], i, 0)),
            pl.BlockSpec((1, C, IBLK), lambda i, b, be, va: (be[b], 0, i)),
        ],
        out_specs=pl.BlockSpec((NSLOT, C), lambda i, b, be, va: (0, 0)),
    )
    ys = pl.pallas_call(
        _moe_sparse_kernel,
        grid_spec=grid_spec,
        out_shape=jax.ShapeDtypeStruct((NSLOT, C), jnp.float32),
        interpret=_INTERPRET,
    )(be_flat, valid_flat, ws2d, xs, W1, W2, Wp)

    y = sc_combine(ys, pos1.reshape(-1), pos2.reshape(-1))
    return y.reshape(Bv, Tv, Cv)


# speedup vs baseline: 1.8270x; 1.8270x over previous
import functools

import jax
import jax.numpy as jnp
from jax import lax
from jax.experimental import pallas as pl
from jax.experimental.pallas import tpu as pltpu
from jax.experimental.pallas import tpu_sc as plsc

N = 2048
C = 1024
E = 8
I = 2816
K = 2
BLK = 512
IBLK = 256
NI = I // IBLK
NSLOT = N * K + E * BLK
NBLK = NSLOT // BLK
NPAD = E * BLK
NWSLOT = NSLOT + NPAD

_NC, _NS = 2, 16
_NW = _NC * _NS
_PPW = (N * K) // _NW
_TPW = N // _NW
_SPW = NSLOT // _NW

_INTERPRET = False


def _dispatch_kernel(x_ref, wg_ref, pos1_ref, pos2_ref, p1_ref, p2_ref,
                     pad_ref, be_ref, valid_ref):
    x = x_ref[...]
    logits = lax.dot_general(x, wg_ref[...], (((1,), (1,)), ((), ())),
                             preferred_element_type=jnp.float32)
    e_iota = lax.broadcasted_iota(jnp.int32, (N, E), 1)
    i1 = jnp.argmax(logits, axis=1)[:, None]
    m1 = jnp.max(logits, axis=1, keepdims=True)
    masked = jnp.where(e_iota == i1, -jnp.inf, logits)
    i2 = jnp.argmax(masked, axis=1)[:, None]
    m2 = jnp.max(masked, axis=1, keepdims=True)
    e2 = jnp.exp(m2 - m1)
    p1_ref[...] = 1.0 / (1.0 + e2)
    p2_ref[...] = e2 / (1.0 + e2)

    M1 = (e_iota == i1).astype(jnp.float32)
    M2 = (e_iota == i2).astype(jnp.float32)
    M12 = jnp.concatenate([M1, M2], axis=1).astype(jnp.bfloat16)
    r_iota = lax.broadcasted_iota(jnp.int32, (N, N), 0)
    c_iota = lax.broadcasted_iota(jnp.int32, (N, N), 1)
    T = (c_iota < r_iota).astype(jnp.bfloat16)
    R12 = lax.dot_general(T, M12, (((1,), (0,)), ((), ())),
                          preferred_element_type=jnp.float32)
    cnt1 = jnp.sum(M1, axis=0, keepdims=True)
    cnt2 = jnp.sum(M2, axis=0, keepdims=True)
    counts = cnt1 + cnt2
    rank1 = R12[:, :E]
    rank2 = cnt1 + R12[:, E:]
    pc = jnp.floor((counts + (BLK - 1)) * (1.0 / BLK)) * BLK
    pb = jnp.zeros((1, E), jnp.float32)
    for k in range(1, E):
        pb = pb + jnp.concatenate(
            [jnp.zeros((1, k), jnp.float32), pc[:, :E - k]], axis=1)
    pos1_ref[...] = jnp.sum(M1 * (pb + rank1), axis=1, keepdims=True)
    pos2_ref[...] = jnp.sum(M2 * (pb + rank2), axis=1, keepdims=True)

    r8 = lax.broadcasted_iota(jnp.int32, (E, BLK), 0)
    j8 = lax.broadcasted_iota(jnp.int32, (E, BLK), 1).astype(jnp.float32)
    padpos = jnp.zeros((E, BLK), jnp.float32)
    for e in range(E):
        base_e = pb[0, e] + counts[0, e]
        npad_e = pc[0, e] - counts[0, e]
        padpos = padpos + jnp.where(
            r8 == e,
            jnp.where(j8 < npad_e, base_e + j8,
                      jnp.float32(NSLOT + e * BLK) + j8),
            0.0)
    pad_ref[...] = padpos

    biota = lax.broadcasted_iota(jnp.int32, (1, 128), 1).astype(jnp.float32)
    beval = jnp.zeros((1, 128), jnp.float32)
    for e in range(1, E):
        beval = beval + (biota >= pb[0, e] * (1.0 / BLK)).astype(jnp.float32)
    be_ref[...] = beval.astype(jnp.int32)
    used_blocks = jnp.sum(pc) * (1.0 / BLK)
    valid_ref[...] = (biota < used_blocks).astype(jnp.int32)


def _sc_dispatch_body(x_hbm, pos_hbm, p_hbm, pad_hbm, xs_hbm, w_hbm,
                      posf_v, posi_v, posr_v, pv_v, padf_v, padi_v, zero_v,
                      rows_a, rows_b, semr_a, semr_b, semw_a, semw_b, sem):
    wid = lax.axis_index("s") * _NC + lax.axis_index("c")
    base = wid * _PPW
    tokbase = (wid % _NS) * _PPW
    npad_w = NPAD // _NW
    pltpu.sync_copy(pos_hbm.at[pl.ds(base, _PPW)], posf_v)
    pltpu.sync_copy(p_hbm.at[pl.ds(base, _PPW)], pv_v)
    pltpu.sync_copy(pad_hbm.at[pl.ds(wid * npad_w, npad_w)], padf_v)
    for j in range(4):
        for i in range(2):
            off = 32 * j + 16 * i
            v = posf_v[pl.ds(off, 16)].astype(jnp.int32)
            posr_v[pl.ds(off, 16)] = v
            posi_v[j, pl.ds(16 * i, 16)] = v
    for i in range(npad_w // 16):
        padi_v[pl.ds(16 * i, 16)] = padf_v[pl.ds(16 * i, 16)].astype(jnp.int32)
        zero_v[pl.ds(16 * i, 16)] = jnp.zeros((16,), jnp.float32)
    cw = pltpu.async_copy(pv_v, w_hbm.at[posr_v], sem)
    cz = pltpu.async_copy(zero_v, w_hbm.at[padi_v], sem)
    bufs = (rows_a, rows_b)
    semr = (semr_a, semr_b)
    semw = (semw_a, semw_b)
    rd = [None] * 4
    wr = [None] * 4
    rd[0] = pltpu.async_copy(x_hbm.at[pl.ds(tokbase, 32), :], rows_a, semr_a)
    for j in range(4):
        rd[j].wait()
        if j + 1 < 4:
            if wr[j - 1] is not None:
                wr[j - 1].wait()
            rd[j + 1] = pltpu.async_copy(
                x_hbm.at[pl.ds(tokbase + 32 * (j + 1), 32), :],
                bufs[(j + 1) % 2], semr[(j + 1) % 2])
        wr[j] = pltpu.async_copy(bufs[j % 2], xs_hbm.at[posi_v.at[j]],
                                 semw[j % 2])
    wr[2].wait()
    wr[3].wait()
    cw.wait()
    cz.wait()


@functools.cache
def _get_sc_kernels():
    sc_mesh = plsc.VectorSubcoreMesh(core_axis_name="c", subcore_axis_name="s",
                                     num_cores=_NC, num_subcores=_NS)
    npad_w = NPAD // _NW
    sc_dispatch = functools.partial(
        pl.kernel,
        out_type=(jax.ShapeDtypeStruct((NSLOT, C), jnp.float32),
                  jax.ShapeDtypeStruct((NWSLOT,), jnp.float32)),
        mesh=sc_mesh,
        scratch_types=[
            pltpu.VMEM((_PPW,), jnp.float32),
            pltpu.VMEM((4, 32), jnp.int32),
            pltpu.VMEM((_PPW,), jnp.int32),
            pltpu.VMEM((_PPW,), jnp.float32),
            pltpu.VMEM((npad_w,), jnp.float32),
            pltpu.VMEM((npad_w,), jnp.int32),
            pltpu.VMEM((npad_w,), jnp.float32),
            pltpu.VMEM((32, C), jnp.float32),
            pltpu.VMEM((32, C), jnp.float32),
            pltpu.SemaphoreType.DMA,
            pltpu.SemaphoreType.DMA,
            pltpu.SemaphoreType.DMA,
            pltpu.SemaphoreType.DMA,
            pltpu.SemaphoreType.DMA,
        ],
    )(_sc_dispatch_body)
    sc_combine = functools.partial(
        pl.kernel,
        out_type=jax.ShapeDtypeStruct((N, C), jnp.float32),
        mesh=sc_mesh,
        scratch_types=[
            pltpu.VMEM((_TPW,), jnp.float32),
            pltpu.VMEM((2, 32), jnp.int32),
            pltpu.VMEM((2, 32), jnp.int32),
            pltpu.VMEM((32, C), jnp.float32),
            pltpu.VMEM((32, C), jnp.float32),
            pltpu.SemaphoreType.DMA,
            pltpu.SemaphoreType.DMA,
        ],
    )(_sc_combine_body)
    return sc_dispatch, sc_combine


def _moe_sparse_kernel(be_ref, valid_ref, ws_ref, xs_ref, w1_ref, w2_ref,
                       wp_ref, out_ref):
    i = pl.program_id(0)
    b = pl.program_id(1)

    @pl.when(valid_ref[b] == 1)
    def _compute():
        rows = xs_ref[...].astype(jnp.bfloat16)
        w1 = w1_ref[0].astype(jnp.bfloat16)
        w2 = w2_ref[0].astype(jnp.bfloat16)
        wp = wp_ref[0].astype(jnp.bfloat16)
        h1 = lax.dot_general(rows, w1, (((1,), (1,)), ((), ())),
                             preferred_element_type=jnp.float32)
        h2 = lax.dot_general(rows, w2, (((1,), (1,)), ((), ())),
                             preferred_element_type=jnp.float32)
        h = h1 * jax.nn.sigmoid(h1) * h2
        hw = (h * ws_ref[...]).astype(jnp.bfloat16)
        pout = lax.dot_general(hw, wp, (((1,), (1,)), ((), ())),
                               preferred_element_type=jnp.float32)
        sl = pl.ds(pl.multiple_of(b * BLK, BLK), BLK)

        @pl.when(i == 0)
        def _first():
            out_ref[sl, :] = pout

        @pl.when(i > 0)
        def _acc():
            out_ref[sl, :] += pout


def _sc_combine_body(ys_hbm, pos1_hbm, pos2_hbm, y_hbm,
                     pf_v, pia_v, pib_v, bufa_v, bufb_v, sema, semb):
    wid = lax.axis_index("s") * _NC + lax.axis_index("c")
    tbase = wid * _TPW
    pltpu.sync_copy(pos1_hbm.at[pl.ds(tbase, _TPW)], pf_v)
    for j in range(2):
        for i in range(2):
            pia_v[j, pl.ds(16 * i, 16)] = (
                pf_v[pl.ds(32 * j + 16 * i, 16)].astype(jnp.int32))
    pltpu.sync_copy(pos2_hbm.at[pl.ds(tbase, _TPW)], pf_v)
    for j in range(2):
        for i in range(2):
            pib_v[j, pl.ds(16 * i, 16)] = (
                pf_v[pl.ds(32 * j + 16 * i, 16)].astype(jnp.int32))
    for j in range(2):
        ca = pltpu.async_copy(ys_hbm.at[pia_v.at[j]], bufa_v, sema)
        cb = pltpu.async_copy(ys_hbm.at[pib_v.at[j]], bufb_v, semb)
        ca.wait()
        cb.wait()

        def row_body(r, carry):
            for v in range(C // 16):
                s = pl.ds(pl.multiple_of(16 * v, 16), 16)
                bufa_v[r, s] = bufa_v[r, s] + bufb_v[r, s]
            return carry

        lax.fori_loop(0, 32, row_body, 0)
        pltpu.sync_copy(bufa_v, y_hbm.at[pl.ds(tbase + 32 * j, 32), :])


def kernel(x, Wg, W1, W2, Wp):
    Bv, Tv, Cv = x.shape
    xf = x.reshape(N, C)

    pos1, pos2, p1, p2, padpos, be, valid = pl.pallas_call(
        _dispatch_kernel,
        out_shape=(
            jax.ShapeDtypeStruct((N, 1), jnp.float32),
            jax.ShapeDtypeStruct((N, 1), jnp.float32),
            jax.ShapeDtypeStruct((N, 1), jnp.float32),
            jax.ShapeDtypeStruct((N, 1), jnp.float32),
            jax.ShapeDtypeStruct((E, BLK), jnp.float32),
            jax.ShapeDtypeStruct((1, 128), jnp.int32),
            jax.ShapeDtypeStruct((1, 128), jnp.int32),
        ),
        interpret=_INTERPRET,
    )(xf, Wg)

    pos_flat = jnp.concatenate([pos1.reshape(-1), pos2.reshape(-1)])
    p_flat = jnp.concatenate([p1.reshape(-1), p2.reshape(-1)])
    pad_flat = padpos.reshape(-1)

    sc_dispatch, sc_combine = _get_sc_kernels()
    xs, wraw = sc_dispatch(xf, pos_flat, p_flat, pad_flat)
    ws2d = wraw[:NSLOT].reshape(NSLOT, 1)
    be_flat = be.reshape(-1)[:NBLK]
    valid_flat = valid.reshape(-1)[:NBLK]

    grid_spec = pltpu.PrefetchScalarGridSpec(
        num_scalar_prefetch=2,
        grid=(NI, NBLK),
        in_specs=[
            pl.BlockSpec((BLK, 1), lambda i, b, be, va: (b, 0)),
            pl.BlockSpec((BLK, C), lambda i, b, be, va: (b, 0)),
            pl.BlockSpec((1, IBLK, C), lambda i, b, be, va: (be[b], i, 0)),
            pl.BlockSpec((1, IBLK, C), lambda i, b, be, va: (be[b], i, 0)),
            pl.BlockSpec((1, C, IBLK), lambda i, b, be, va: (be[b], 0, i)),
        ],
        out_specs=pl.BlockSpec((NSLOT, C), lambda i, b, be, va: (0, 0)),
    )
    ys = pl.pallas_call(
        _moe_sparse_kernel,
        grid_spec=grid_spec,
        out_shape=jax.ShapeDtypeStruct((NSLOT, C), jnp.float32),
        interpret=_INTERPRET,
    )(be_flat, valid_flat, ws2d, xs, W1, W2, Wp)

    y = sc_combine(ys, pos1.reshape(-1), pos2.reshape(-1))
    return y.reshape(Bv, Tv, Cv)


# speedup vs baseline: 1.9574x; 1.0713x over previous
import functools

import jax
import jax.numpy as jnp
from jax import lax
from jax.experimental import pallas as pl
from jax.experimental.pallas import tpu as pltpu
from jax.experimental.pallas import tpu_sc as plsc

N = 2048
C = 1024
E = 8
I = 2816
K = 2
BLK = 512
IBLK = 256
NI = I // IBLK
NSLOT = N * K + E * BLK
NBLK = NSLOT // BLK
NPAD = E * BLK
NWSLOT = NSLOT + NPAD

_NC, _NS = 2, 16
_NW = _NC * _NS
_PPW = (N * K) // _NW
_TPW = N // _NW
_SPW = NSLOT // _NW

_INTERPRET = False


def _dispatch_kernel(x_ref, wg_ref, pos1_ref, pos2_ref, p1_ref, p2_ref,
                     pad_ref, be_ref, valid_ref):
    x = x_ref[...]
    logits = lax.dot_general(x, wg_ref[...], (((1,), (1,)), ((), ())),
                             preferred_element_type=jnp.float32)
    e_iota = lax.broadcasted_iota(jnp.int32, (N, E), 1)
    i1 = jnp.argmax(logits, axis=1)[:, None]
    m1 = jnp.max(logits, axis=1, keepdims=True)
    masked = jnp.where(e_iota == i1, -jnp.inf, logits)
    i2 = jnp.argmax(masked, axis=1)[:, None]
    m2 = jnp.max(masked, axis=1, keepdims=True)
    e2 = jnp.exp(m2 - m1)
    p1_ref[...] = 1.0 / (1.0 + e2)
    p2_ref[...] = e2 / (1.0 + e2)

    M1 = (e_iota == i1).astype(jnp.float32)
    M2 = (e_iota == i2).astype(jnp.float32)
    M12 = jnp.concatenate([M1, M2], axis=1).astype(jnp.bfloat16)
    r_iota = lax.broadcasted_iota(jnp.int32, (N, N), 0)
    c_iota = lax.broadcasted_iota(jnp.int32, (N, N), 1)
    T = (c_iota < r_iota).astype(jnp.bfloat16)
    R12 = lax.dot_general(T, M12, (((1,), (0,)), ((), ())),
                          preferred_element_type=jnp.float32)
    cnt1 = jnp.sum(M1, axis=0, keepdims=True)
    cnt2 = jnp.sum(M2, axis=0, keepdims=True)
    counts = cnt1 + cnt2
    rank1 = R12[:, :E]
    rank2 = cnt1 + R12[:, E:]
    pc = jnp.floor((counts + (BLK - 1)) * (1.0 / BLK)) * BLK
    pb = jnp.zeros((1, E), jnp.float32)
    for k in range(1, E):
        pb = pb + jnp.concatenate(
            [jnp.zeros((1, k), jnp.float32), pc[:, :E - k]], axis=1)
    pos1_ref[...] = jnp.sum(M1 * (pb + rank1), axis=1, keepdims=True)
    pos2_ref[...] = jnp.sum(M2 * (pb + rank2), axis=1, keepdims=True)

    r8 = lax.broadcasted_iota(jnp.int32, (E, BLK), 0)
    j8 = lax.broadcasted_iota(jnp.int32, (E, BLK), 1).astype(jnp.float32)
    padpos = jnp.zeros((E, BLK), jnp.float32)
    for e in range(E):
        base_e = pb[0, e] + counts[0, e]
        npad_e = pc[0, e] - counts[0, e]
        padpos = padpos + jnp.where(
            r8 == e,
            jnp.where(j8 < npad_e, base_e + j8,
                      jnp.float32(NSLOT + e * BLK) + j8),
            0.0)
    pad_ref[...] = padpos

    biota = lax.broadcasted_iota(jnp.int32, (1, 128), 1).astype(jnp.float32)
    beval = jnp.zeros((1, 128), jnp.float32)
    for e in range(1, E):
        beval = beval + (biota >= pb[0, e] * (1.0 / BLK)).astype(jnp.float32)
    be_ref[...] = beval.astype(jnp.int32)
    used_blocks = jnp.sum(pc) * (1.0 / BLK)
    valid_ref[...] = (biota < used_blocks).astype(jnp.int32)


def _sc_dispatch_body(x_hbm, pos_hbm, p_hbm, pad_hbm, xs_hbm, w_hbm,
                      posf_v, posi_v, posr_v, pv_v, padf_v, padi_v, zero_v,
                      rows_a, rows_b, semr_a, semr_b, semw_a, semw_b, sem):
    wid = lax.axis_index("s") * _NC + lax.axis_index("c")
    base = wid * _PPW
    tokbase = (wid % _NS) * _PPW
    npad_w = NPAD // _NW
    pltpu.sync_copy(pos_hbm.at[pl.ds(base, _PPW)], posf_v)
    pltpu.sync_copy(p_hbm.at[pl.ds(base, _PPW)], pv_v)
    pltpu.sync_copy(pad_hbm.at[pl.ds(wid * npad_w, npad_w)], padf_v)
    for j in range(4):
        for i in range(2):
            off = 32 * j + 16 * i
            v = posf_v[pl.ds(off, 16)].astype(jnp.int32)
            posr_v[pl.ds(off, 16)] = v
            posi_v[j, pl.ds(16 * i, 16)] = v
    for i in range(npad_w // 16):
        padi_v[pl.ds(16 * i, 16)] = padf_v[pl.ds(16 * i, 16)].astype(jnp.int32)
        zero_v[pl.ds(16 * i, 16)] = jnp.zeros((16,), jnp.float32)
    cw = pltpu.async_copy(pv_v, w_hbm.at[posr_v], sem)
    cz = pltpu.async_copy(zero_v, w_hbm.at[padi_v], sem)
    bufs = (rows_a, rows_b)
    semr = (semr_a, semr_b)
    semw = (semw_a, semw_b)
    rd = [None] * 4
    wr = [None] * 4
    rd[0] = pltpu.async_copy(x_hbm.at[pl.ds(tokbase, 32), :], rows_a, semr_a)
    for j in range(4):
        rd[j].wait()
        if j + 1 < 4:
            if wr[j - 1] is not None:
                wr[j - 1].wait()
            rd[j + 1] = pltpu.async_copy(
                x_hbm.at[pl.ds(tokbase + 32 * (j + 1), 32), :],
                bufs[(j + 1) % 2], semr[(j + 1) % 2])
        wr[j] = pltpu.async_copy(bufs[j % 2], xs_hbm.at[posi_v.at[j]],
                                 semw[j % 2])
    wr[2].wait()
    wr[3].wait()
    cw.wait()
    cz.wait()


@functools.cache
def _get_sc_kernels():
    sc_mesh = plsc.VectorSubcoreMesh(core_axis_name="c", subcore_axis_name="s",
                                     num_cores=_NC, num_subcores=_NS)
    npad_w = NPAD // _NW
    sc_dispatch = functools.partial(
        pl.kernel,
        out_type=(jax.ShapeDtypeStruct((NSLOT, C), jnp.float32),
                  jax.ShapeDtypeStruct((NWSLOT,), jnp.float32)),
        mesh=sc_mesh,
        scratch_types=[
            pltpu.VMEM((_PPW,), jnp.float32),
            pltpu.VMEM((4, 32), jnp.int32),
            pltpu.VMEM((_PPW,), jnp.int32),
            pltpu.VMEM((_PPW,), jnp.float32),
            pltpu.VMEM((npad_w,), jnp.float32),
            pltpu.VMEM((npad_w,), jnp.int32),
            pltpu.VMEM((npad_w,), jnp.float32),
            pltpu.VMEM((32, C), jnp.float32),
            pltpu.VMEM((32, C), jnp.float32),
            pltpu.SemaphoreType.DMA,
            pltpu.SemaphoreType.DMA,
            pltpu.SemaphoreType.DMA,
            pltpu.SemaphoreType.DMA,
            pltpu.SemaphoreType.DMA,
        ],
    )(_sc_dispatch_body)
    sc_combine = functools.partial(
        pl.kernel,
        out_type=jax.ShapeDtypeStruct((N, C), jnp.float32),
        mesh=sc_mesh,
        scratch_types=[
            pltpu.VMEM((_TPW,), jnp.float32),
            pltpu.VMEM((2, 32), jnp.int32),
            pltpu.VMEM((2, 32), jnp.int32),
            pltpu.VMEM((32, C), jnp.float32),
            pltpu.VMEM((32, C), jnp.float32),
            pltpu.SemaphoreType.DMA,
            pltpu.SemaphoreType.DMA,
        ],
    )(_sc_combine_body)
    return sc_dispatch, sc_combine


def _cast_kernel(xs_ref, o_ref):
    o_ref[...] = xs_ref[...].astype(jnp.bfloat16)


def _moe_sparse_kernel(be_ref, valid_ref, ws_ref, xs_ref, w1_ref, w2_ref,
                       wp_ref, out_ref):
    i = pl.program_id(0)
    b = pl.program_id(1)

    @pl.when(valid_ref[b] == 1)
    def _compute():
        rows = xs_ref[pl.ds(pl.multiple_of(b * BLK, BLK), BLK), :]
        w1 = w1_ref[0].astype(jnp.bfloat16)
        w2 = w2_ref[0].astype(jnp.bfloat16)
        wp = wp_ref[0].astype(jnp.bfloat16)
        h1 = lax.dot_general(rows, w1, (((1,), (1,)), ((), ())),
                             preferred_element_type=jnp.float32)
        h2 = lax.dot_general(rows, w2, (((1,), (1,)), ((), ())),
                             preferred_element_type=jnp.float32)
        h = h1 * jax.nn.sigmoid(h1) * h2
        hw = (h * ws_ref[...]).astype(jnp.bfloat16)
        pout = lax.dot_general(hw, wp, (((1,), (1,)), ((), ())),
                               preferred_element_type=jnp.float32)
        sl = pl.ds(pl.multiple_of(b * BLK, BLK), BLK)

        @pl.when(i == 0)
        def _first():
            out_ref[sl, :] = pout

        @pl.when(i > 0)
        def _acc():
            out_ref[sl, :] += pout


def _sc_combine_body(ys_hbm, pos1_hbm, pos2_hbm, y_hbm,
                     pf_v, pia_v, pib_v, bufa_v, bufb_v, sema, semb):
    wid = lax.axis_index("s") * _NC + lax.axis_index("c")
    tbase = wid * _TPW
    pltpu.sync_copy(pos1_hbm.at[pl.ds(tbase, _TPW)], pf_v)
    for j in range(2):
        for i in range(2):
            pia_v[j, pl.ds(16 * i, 16)] = (
                pf_v[pl.ds(32 * j + 16 * i, 16)].astype(jnp.int32))
    pltpu.sync_copy(pos2_hbm.at[pl.ds(tbase, _TPW)], pf_v)
    for j in range(2):
        for i in range(2):
            pib_v[j, pl.ds(16 * i, 16)] = (
                pf_v[pl.ds(32 * j + 16 * i, 16)].astype(jnp.int32))
    for j in range(2):
        ca = pltpu.async_copy(ys_hbm.at[pia_v.at[j]], bufa_v, sema)
        cb = pltpu.async_copy(ys_hbm.at[pib_v.at[j]], bufb_v, semb)
        ca.wait()
        cb.wait()

        def row_body(r, carry):
            for v in range(C // 16):
                s = pl.ds(pl.multiple_of(16 * v, 16), 16)
                bufa_v[r, s] = bufa_v[r, s] + bufb_v[r, s]
            return carry

        lax.fori_loop(0, 32, row_body, 0)
        pltpu.sync_copy(bufa_v, y_hbm.at[pl.ds(tbase + 32 * j, 32), :])


def kernel(x, Wg, W1, W2, Wp):
    Bv, Tv, Cv = x.shape
    xf = x.reshape(N, C)

    pos1, pos2, p1, p2, padpos, be, valid = pl.pallas_call(
        _dispatch_kernel,
        out_shape=(
            jax.ShapeDtypeStruct((N, 1), jnp.float32),
            jax.ShapeDtypeStruct((N, 1), jnp.float32),
            jax.ShapeDtypeStruct((N, 1), jnp.float32),
            jax.ShapeDtypeStruct((N, 1), jnp.float32),
            jax.ShapeDtypeStruct((E, BLK), jnp.float32),
            jax.ShapeDtypeStruct((1, 128), jnp.int32),
            jax.ShapeDtypeStruct((1, 128), jnp.int32),
        ),
        interpret=_INTERPRET,
    )(xf, Wg)

    pos_flat = jnp.concatenate([pos1.reshape(-1), pos2.reshape(-1)])
    p_flat = jnp.concatenate([p1.reshape(-1), p2.reshape(-1)])
    pad_flat = padpos.reshape(-1)

    sc_dispatch, sc_combine = _get_sc_kernels()
    xs, wraw = sc_dispatch(xf, pos_flat, p_flat, pad_flat)
    ws2d = wraw[:NSLOT].reshape(NSLOT, 1)
    be_flat = be.reshape(-1)[:NBLK]
    valid_flat = valid.reshape(-1)[:NBLK]

    xsb = pl.pallas_call(
        _cast_kernel,
        grid=(NBLK,),
        in_specs=[pl.BlockSpec((BLK, C), lambda b: (b, 0))],
        out_specs=pl.BlockSpec((BLK, C), lambda b: (b, 0)),
        out_shape=jax.ShapeDtypeStruct((NSLOT, C), jnp.bfloat16),
        interpret=_INTERPRET,
    )(xs)

    grid_spec = pltpu.PrefetchScalarGridSpec(
        num_scalar_prefetch=2,
        grid=(NI, NBLK),
        in_specs=[
            pl.BlockSpec((BLK, 1), lambda i, b, be, va: (b, 0)),
            pl.BlockSpec((NSLOT, C), lambda i, b, be, va: (0, 0)),
            pl.BlockSpec((1, IBLK, C), lambda i, b, be, va: (be[b], i, 0)),
            pl.BlockSpec((1, IBLK, C), lambda i, b, be, va: (be[b], i, 0)),
            pl.BlockSpec((1, C, IBLK), lambda i, b, be, va: (be[b], 0, i)),
        ],
        out_specs=pl.BlockSpec((NSLOT, C), lambda i, b, be, va: (0, 0)),
    )
    ys = pl.pallas_call(
        _moe_sparse_kernel,
        grid_spec=grid_spec,
        out_shape=jax.ShapeDtypeStruct((NSLOT, C), jnp.float32),
        interpret=_INTERPRET,
    )(be_flat, valid_flat, ws2d, xsb, W1, W2, Wp)

    y = sc_combine(ys, pos1.reshape(-1), pos2.reshape(-1))
    return y.reshape(Bv, Tv, Cv)
